# baseline (speedup 1.0000x reference)
import functools
import math

import jax
import jax.numpy as jnp
from jax import lax
from jax.experimental import pallas as pl
from jax.experimental.pallas import tpu as pltpu
from jax.experimental.pallas import tpu_sc as plsc

_B = 16384
_RV = 512
_AV = 49
_AVP = 64
_ED = 32
_NA = 6
_NC = 2
_NS = 16
_NW = _NC * _NS
_NSM = 4
_SPS = 416
_SPL = 544
_SVS = _SPS // 16
_SVL = _SPL // 16

_ATAN_C = (
    9.9987876415e-01, -3.3040556312e-01, 1.8041267991e-01, -8.5408307612e-02,
    2.0931812003e-02,
)
_LOG_C = (
    1.0000143716e+00, -4.9984405492e-01, 3.3224232786e-01, -2.5487297867e-01,
    2.2325265353e-01, -1.4230193465e-01,
)


def _sc_body(p_hbm, rt_hbm, at_hbm, w_hbm, b_hbm, out_hbm,
             p_v, rt_v, at_v, w_v, b_v, chunk_v, idx_v,
             pr_sh, pa_sh, pr_v, pa_v, out_v,
             sem_p, sem_rt, sem_w, sem_b):
    c = lax.axis_index("c")
    s = lax.axis_index("s")
    small = s < 4
    base = c * (_B // _NC) + jnp.where(small, s * _SPS, _NSM * _SPS
                                       + (s - _NSM) * _SPL)
    iota = lax.broadcasted_iota(jnp.int32, (16,), 0)

    cp_p = pltpu.async_copy(p_hbm.at[:, pl.ds(base, _SPL)], p_v, sem_p)
    cp_rt = pltpu.async_copy(rt_hbm.at[pl.ds(s * 32, 32)], rt_v, sem_rt)
    cp_w = pltpu.async_copy(w_hbm, w_v, sem_w)
    cp_b = pltpu.async_copy(b_hbm, b_v.at[pl.ds(0, _NA)], sem_b)
    cp_w.wait()
    cp_b.wait()
    cp_rt.wait()

    acols = [jnp.full((16,), a, jnp.int32) for a in range(_NA)]
    bvec = b_v[pl.ds(0, 16)]

    init = tuple(jnp.full((16,), bvec[a]) for a in range(_NA))

    @plsc.parallel_loop(0, _ED, 1, unroll=2, carry=(init, init))
    def proj_body(d, accs):
        dvec = jnp.full((16,), d, jnp.int32)
        col0 = plsc.load_gather(rt_v, [iota, dvec])
        col1 = plsc.load_gather(rt_v, [iota + 16, dvec])
        out = []
        for a in range(_NA):
            wv = plsc.load_gather(w_v, [acols[a], dvec])
            out.append((accs[0][a] + col0 * wv, accs[1][a] + col1 * wv))
        return tuple(zip(*out))

    acc0, acc1 = proj_body
    for a in range(_NA):
        chunk_v[a, pl.ds(0, 16)] = acc0[a]
        chunk_v[a, pl.ds(16, 16)] = acc1[a]
    for a in range(_NA):
        pltpu.sync_copy(chunk_v.at[a], pr_sh.at[a, pl.ds(s * 32, 32)])

    @pl.when(s < 4)
    def _angle_proj():
        astart = jnp.where(s == 3, 33, s * 16)
        loc = jnp.where(s == 3, jnp.full((16,), 15, jnp.int32), iota)
        pltpu.sync_copy(at_hbm.at[pl.ds(astart, 16)], at_v)

        zero6 = tuple(jnp.zeros((16,), jnp.float32) for _ in range(_NA))

        @plsc.parallel_loop(0, _ED, 1, unroll=2, carry=zero6)
        def aproj_body(d, accs):
            dvec = jnp.full((16,), d, jnp.int32)
            col = plsc.load_gather(at_v, [loc, dvec])
            dvec2 = dvec + _ED
            return tuple(
                accs[a] + col * plsc.load_gather(w_v, [acols[a], dvec2])
                for a in range(_NA))

        aacc = aproj_body
        for a in range(_NA):
            chunk_v[a, pl.ds(0, 16)] = aacc[a]
        for a in range(_NA):
            pltpu.sync_copy(chunk_v.at[a, pl.ds(0, 16)],
                            pa_sh.at[a, pl.ds(s * 16, 16)])

    cp_p.wait()

    f32 = jnp.float32
    atanc = [jnp.full((16,), f32(v)) for v in _ATAN_C]
    logc = [jnp.full((16,), f32(v)) for v in _LOG_C]
    c_tiny = jnp.full((16,), f32(1e-35))
    c_magic = jnp.full((16,), 0x5F3759DF, jnp.int32)
    c_15 = jnp.full((16,), f32(1.5))
    c_half = jnp.full((16,), f32(0.5))
    c_02 = jnp.full((16,), f32(0.2))
    c_hpi = jnp.full((16,), f32(math.pi / 2.0))
    c_pi = jnp.full((16,), f32(math.pi))
    c_r2d = jnp.full((16,), f32(180.0 / math.pi))
    c_90 = jnp.full((16,), f32(90.0))
    c_360 = jnp.full((16,), f32(360.0))
    c_i360 = jnp.full((16,), f32(1.0 / 360.0))
    c_i75 = jnp.full((16,), f32(1.0 / 7.5))
    c_zero = jnp.zeros((16,), f32)
    c_one = jnp.full((16,), f32(1.0))
    c_ln2 = jnp.full((16,), f32(math.log(2.0)))
    c_sq2 = jnp.full((16,), f32(math.sqrt(2.0)))
    c_127 = jnp.full((16,), 127, jnp.int32)
    c_mant = jnp.full((16,), 0x7FFFFF, jnp.int32)
    c_expo = jnp.full((16,), 127 << 23, jnp.int32)
    c_63 = jnp.full((16,), 63, jnp.int32)

    def one_idx(o):
        ax = p_v[0, pl.ds(o, 16)]
        az = p_v[1, pl.ds(o, 16)]
        pose = p_v[2, pl.ds(o, 16)]
        gx = p_v[3, pl.ds(o, 16)]
        gz = p_v[4, pl.ds(o, 16)]
        dx = gx - ax
        dz = gz - az
        d2 = jnp.maximum(dx * dx + dz * dz, c_tiny)

        sb = lax.bitcast_convert_type(d2, jnp.int32)
        y = lax.bitcast_convert_type(c_magic - (sb >> 1), jnp.float32)
        d2h = d2 * c_half
        y = y * (c_15 - d2h * y * y)
        y = y * (c_15 - d2h * y * y)
        x = d2 * y
        r_idx = (x * c_02).astype(jnp.int32)

        axa = jnp.abs(dx)
        aya = jnp.abs(dz)
        swap = aya > axa
        num = jnp.where(swap, axa, aya)
        den = jnp.where(swap, aya, axa)
        t = num / den
        t = jnp.where(den == c_zero, c_zero, t)
        u = t * t
        p = atanc[4]
        for k in range(3, -1, -1):
            p = p * u + atanc[k]
        p = p * t
        r = jnp.where(swap, c_hpi - p, p)
        r = jnp.where(dx < c_zero, c_pi - r, r)
        r = jnp.where(dz < c_zero, -r, r)

        diff = c_90 - r * c_r2d - pose
        q = diff * c_i360
        qt = q.astype(jnp.int32).astype(jnp.float32)
        m = diff - qt * c_360
        m = jnp.where(m < c_zero, m + c_360, m)
        m = jnp.where(m >= c_360, m - c_360, m)
        t_idx = jnp.minimum((m * c_i75).astype(jnp.int32), c_63)

        idx_v[0, pl.ds(o, 16)] = r_idx
        idx_v[1, pl.ds(o, 16)] = t_idx

    @plsc.parallel_loop(0, _SVS, 1, unroll=2)
    def body_a(i):
        one_idx(i * 16)

    @pl.when(jnp.logical_not(small))
    def _rest_a():
        @plsc.parallel_loop(_SVS, _SVL, 1, unroll=2)
        def body_a2(i):
            one_idx(i * 16)

    plsc.subcore_barrier()
    pltpu.sync_copy(pr_sh, pr_v)
    pltpu.sync_copy(pa_sh, pa_v)

    def one_out(o):
        r_idx = idx_v[0, pl.ds(o, 16)]
        t_idx = idx_v[1, pl.ds(o, 16)]
        logits = []
        for a in range(_NA):
            lr = plsc.load_gather(pr_v, [acols[a], r_idx])
            la = plsc.load_gather(pa_v, [acols[a], t_idx])
            logits.append(lr + la)
        es = [jnp.exp(v) for v in logits]
        tot = ((es[0] + es[1]) + (es[2] + es[3])) + (es[4] + es[5])

        tb = lax.bitcast_convert_type(tot, jnp.int32)
        e = (tb >> 23) - c_127
        mb = (tb & c_mant) | c_expo
        mf = lax.bitcast_convert_type(mb, jnp.float32)
        big = mf > c_sq2
        mf = jnp.where(big, mf * c_half, mf)
        e = e + big.astype(jnp.int32)
        z = mf - c_one
        qq = logc[5]
        for k in range(4, -1, -1):
            qq = qq * z + logc[k]
        lse = e.astype(jnp.float32) * c_ln2 + qq * z

        for a in range(_NA):
            out_v[a, pl.ds(o, 16)] = logits[a] - lse

    @plsc.parallel_loop(0, _SVS, 1, unroll=2)
    def body_b(i):
        one_out(i * 16)

    @pl.when(jnp.logical_not(small))
    def _rest_b():
        @plsc.parallel_loop(_SVS, _SVL, 1, unroll=2)
        def body_b2(i):
            one_out(i * 16)

    @pl.when(small)
    def _out_small():
        pltpu.sync_copy(out_v.at[:, pl.ds(0, _SPS)],
                        out_hbm.at[:, pl.ds(base, _SPS)])

    @pl.when(jnp.logical_not(small))
    def _out_large():
        pltpu.sync_copy(out_v, out_hbm.at[:, pl.ds(base, _SPL)])


@functools.partial(
    pl.kernel,
    out_type=jax.ShapeDtypeStruct((_NA, _B), jnp.float32),
    mesh=plsc.VectorSubcoreMesh(core_axis_name="c", subcore_axis_name="s",
                                num_cores=_NC, num_subcores=_NS),
    compiler_params=pltpu.CompilerParams(needs_layout_passes=False,
                                         use_tc_tiling_on_sc=False),
    scratch_types=[
        pltpu.VMEM((5, _SPL), jnp.float32),
        pltpu.VMEM((32, _ED), jnp.float32),
        pltpu.VMEM((16, _ED), jnp.float32),
        pltpu.VMEM((_NA, 2 * _ED), jnp.float32),
        pltpu.VMEM((16,), jnp.float32),
        pltpu.VMEM((_NA, 32), jnp.float32),
        pltpu.VMEM((2, _SPL), jnp.int32),
        pltpu.VMEM_SHARED((_NA, _RV), jnp.float32),
        pltpu.VMEM_SHARED((_NA, _AVP), jnp.float32),
        pltpu.VMEM((_NA, _RV), jnp.float32),
        pltpu.VMEM((_NA, _AVP), jnp.float32),
        pltpu.VMEM((_NA, _SPL), jnp.float32),
        pltpu.SemaphoreType.DMA,
        pltpu.SemaphoreType.DMA,
        pltpu.SemaphoreType.DMA,
        pltpu.SemaphoreType.DMA,
    ],
)
def _sc_kernel(p_hbm, rt_hbm, at_hbm, w_hbm, b_hbm, out_hbm,
               p_v, rt_v, at_v, w_v, b_v, chunk_v, idx_v,
               pr_sh, pa_sh, pr_v, pa_v, out_v,
               sem_p, sem_rt, sem_w, sem_b):
    _sc_body(p_hbm, rt_hbm, at_hbm, w_hbm, b_hbm, out_hbm,
             p_v, rt_v, at_v, w_v, b_v, chunk_v, idx_v,
             pr_sh, pa_sh, pr_v, pa_v, out_v,
             sem_p, sem_rt, sem_w, sem_b)


def kernel(agent_positions, goal_positions, radius_table, angle_table, W, b):
    pos = jnp.concatenate([agent_positions, goal_positions], axis=1).T
    out = _sc_kernel(pos, radius_table, angle_table, W, b)
    return out.T


# speedup vs baseline: 1.0176x; 1.0176x over previous
import functools
import math

import jax
import jax.numpy as jnp
from jax import lax
from jax.experimental import pallas as pl
from jax.experimental.pallas import tpu as pltpu
from jax.experimental.pallas import tpu_sc as plsc

_B = 16384
_RV = 512
_AV = 49
_AVP = 64
_ED = 32
_NA = 6
_NC = 2
_NS = 16
_NW = _NC * _NS
_SPT = _B // _NW
_SV = _SPT // 16

_ATAN_C = (
    9.9987876415e-01, -3.3040556312e-01, 1.8041267991e-01, -8.5408307612e-02,
    2.0931812003e-02,
)
_LOG_C = (
    1.0000143716e+00, -4.9984405492e-01, 3.3224232786e-01, -2.5487297867e-01,
    2.2325265353e-01, -1.4230193465e-01,
)


def _sc_body(p_hbm, rt_hbm, at_hbm, w_hbm, b_hbm, out_hbm,
             p_v, rt_v, at_v, w_v, b_v, chunk_v, idx_v,
             pr_sh, pa_sh, pr_v, pa_v, out_v,
             sem_p, sem_rt, sem_w, sem_b):
    c = lax.axis_index("c")
    s = lax.axis_index("s")
    wid = s * _NC + c
    base = wid * _SPT
    iota = lax.broadcasted_iota(jnp.int32, (16,), 0)

    cp_p = pltpu.async_copy(p_hbm.at[:, pl.ds(base, _SPT)], p_v, sem_p)
    cp_rt = pltpu.async_copy(rt_hbm.at[pl.ds(s * 32, 32)], rt_v, sem_rt)
    cp_w = pltpu.async_copy(w_hbm, w_v, sem_w)
    cp_b = pltpu.async_copy(b_hbm, b_v.at[pl.ds(0, _NA)], sem_b)
    cp_w.wait()
    cp_b.wait()
    cp_rt.wait()

    acols = [jnp.full((16,), a, jnp.int32) for a in range(_NA)]
    bvec = b_v[pl.ds(0, 16)]

    init = tuple(jnp.full((16,), bvec[a]) for a in range(_NA))

    @plsc.parallel_loop(0, _ED, 1, unroll=2, carry=(init, init))
    def proj_body(d, accs):
        dvec = jnp.full((16,), d, jnp.int32)
        col0 = plsc.load_gather(rt_v, [iota, dvec])
        col1 = plsc.load_gather(rt_v, [iota + 16, dvec])
        out = []
        for a in range(_NA):
            wv = plsc.load_gather(w_v, [acols[a], dvec])
            out.append((accs[0][a] + col0 * wv, accs[1][a] + col1 * wv))
        return tuple(zip(*out))

    acc0, acc1 = proj_body
    for a in range(_NA):
        chunk_v[a, pl.ds(0, 16)] = acc0[a]
        chunk_v[a, pl.ds(16, 16)] = acc1[a]
    for a in range(_NA):
        pltpu.sync_copy(chunk_v.at[a], pr_sh.at[a, pl.ds(s * 32, 32)])

    @pl.when(s < 4)
    def _angle_proj():
        astart = jnp.where(s == 3, 33, s * 16)
        loc = jnp.where(s == 3, jnp.full((16,), 15, jnp.int32), iota)
        pltpu.sync_copy(at_hbm.at[pl.ds(astart, 16)], at_v)

        zero6 = tuple(jnp.zeros((16,), jnp.float32) for _ in range(_NA))

        @plsc.parallel_loop(0, _ED, 1, unroll=2, carry=zero6)
        def aproj_body(d, accs):
            dvec = jnp.full((16,), d, jnp.int32)
            col = plsc.load_gather(at_v, [loc, dvec])
            dvec2 = dvec + _ED
            return tuple(
                accs[a] + col * plsc.load_gather(w_v, [acols[a], dvec2])
                for a in range(_NA))

        aacc = aproj_body
        for a in range(_NA):
            chunk_v[a, pl.ds(0, 16)] = aacc[a]
        for a in range(_NA):
            pltpu.sync_copy(chunk_v.at[a, pl.ds(0, 16)],
                            pa_sh.at[a, pl.ds(s * 16, 16)])

    cp_p.wait()

    f32 = jnp.float32
    atanc = [jnp.full((16,), f32(v)) for v in _ATAN_C]
    logc = [jnp.full((16,), f32(v)) for v in _LOG_C]
    c_tiny = jnp.full((16,), f32(1e-35))
    c_magic = jnp.full((16,), 0x5F3759DF, jnp.int32)
    c_15 = jnp.full((16,), f32(1.5))
    c_half = jnp.full((16,), f32(0.5))
    c_02 = jnp.full((16,), f32(0.2))
    c_hpi = jnp.full((16,), f32(math.pi / 2.0))
    c_pi = jnp.full((16,), f32(math.pi))
    c_r2d = jnp.full((16,), f32(180.0 / math.pi))
    c_90 = jnp.full((16,), f32(90.0))
    c_360 = jnp.full((16,), f32(360.0))
    c_i360 = jnp.full((16,), f32(1.0 / 360.0))
    c_i75 = jnp.full((16,), f32(1.0 / 7.5))
    c_zero = jnp.zeros((16,), f32)
    c_one = jnp.full((16,), f32(1.0))
    c_ln2 = jnp.full((16,), f32(math.log(2.0)))
    c_sq2 = jnp.full((16,), f32(math.sqrt(2.0)))
    c_127 = jnp.full((16,), 127, jnp.int32)
    c_mant = jnp.full((16,), 0x7FFFFF, jnp.int32)
    c_expo = jnp.full((16,), 127 << 23, jnp.int32)
    c_63 = jnp.full((16,), 63, jnp.int32)

    def one_idx(o):
        ax = p_v[0, pl.ds(o, 16)]
        az = p_v[1, pl.ds(o, 16)]
        pose = p_v[2, pl.ds(o, 16)]
        gx = p_v[3, pl.ds(o, 16)]
        gz = p_v[4, pl.ds(o, 16)]
        dx = gx - ax
        dz = gz - az
        d2 = jnp.maximum(dx * dx + dz * dz, c_tiny)

        sb = lax.bitcast_convert_type(d2, jnp.int32)
        y = lax.bitcast_convert_type(c_magic - (sb >> 1), jnp.float32)
        d2h = d2 * c_half
        y = y * (c_15 - d2h * y * y)
        y = y * (c_15 - d2h * y * y)
        x = d2 * y
        r_idx = (x * c_02).astype(jnp.int32)

        axa = jnp.abs(dx)
        aya = jnp.abs(dz)
        swap = aya > axa
        num = jnp.where(swap, axa, aya)
        den = jnp.where(swap, aya, axa)
        t = num / den
        t = jnp.where(den == c_zero, c_zero, t)
        u = t * t
        p = atanc[4]
        for k in range(3, -1, -1):
            p = p * u + atanc[k]
        p = p * t
        r = jnp.where(swap, c_hpi - p, p)
        r = jnp.where(dx < c_zero, c_pi - r, r)
        r = jnp.where(dz < c_zero, -r, r)

        diff = c_90 - r * c_r2d - pose
        q = diff * c_i360
        qt = q.astype(jnp.int32).astype(jnp.float32)
        m = diff - qt * c_360
        m = jnp.where(m < c_zero, m + c_360, m)
        m = jnp.where(m >= c_360, m - c_360, m)
        t_idx = jnp.minimum((m * c_i75).astype(jnp.int32), c_63)

        idx_v[0, pl.ds(o, 16)] = r_idx
        idx_v[1, pl.ds(o, 16)] = t_idx

    @plsc.parallel_loop(0, _SV, 1, unroll=4)
    def body_a(i):
        one_idx(i * 16)

    plsc.subcore_barrier()
    pltpu.sync_copy(pr_sh, pr_v)
    pltpu.sync_copy(pa_sh, pa_v)

    def one_out(o):
        r_idx = idx_v[0, pl.ds(o, 16)]
        t_idx = idx_v[1, pl.ds(o, 16)]
        logits = []
        for a in range(_NA):
            lr = plsc.load_gather(pr_v, [acols[a], r_idx])
            la = plsc.load_gather(pa_v, [acols[a], t_idx])
            logits.append(lr + la)
        es = [jnp.exp(v) for v in logits]
        tot = ((es[0] + es[1]) + (es[2] + es[3])) + (es[4] + es[5])

        tb = lax.bitcast_convert_type(tot, jnp.int32)
        e = (tb >> 23) - c_127
        mb = (tb & c_mant) | c_expo
        mf = lax.bitcast_convert_type(mb, jnp.float32)
        big = mf > c_sq2
        mf = jnp.where(big, mf * c_half, mf)
        e = e + big.astype(jnp.int32)
        z = mf - c_one
        qq = logc[5]
        for k in range(4, -1, -1):
            qq = qq * z + logc[k]
        lse = e.astype(jnp.float32) * c_ln2 + qq * z

        for a in range(_NA):
            out_v[a, pl.ds(o, 16)] = logits[a] - lse

    @plsc.parallel_loop(0, _SV, 1, unroll=4)
    def body_b(i):
        one_out(i * 16)
    pltpu.sync_copy(out_v, out_hbm.at[:, pl.ds(base, _SPT)])


@functools.partial(
    pl.kernel,
    out_type=jax.ShapeDtypeStruct((_NA, _B), jnp.float32),
    mesh=plsc.VectorSubcoreMesh(core_axis_name="c", subcore_axis_name="s",
                                num_cores=_NC, num_subcores=_NS),
    compiler_params=pltpu.CompilerParams(needs_layout_passes=False,
                                         use_tc_tiling_on_sc=False),
    scratch_types=[
        pltpu.VMEM((5, _SPT), jnp.float32),
        pltpu.VMEM((32, _ED), jnp.float32),
        pltpu.VMEM((16, _ED), jnp.float32),
        pltpu.VMEM((_NA, 2 * _ED), jnp.float32),
        pltpu.VMEM((16,), jnp.float32),
        pltpu.VMEM((_NA, 32), jnp.float32),
        pltpu.VMEM((2, _SPT), jnp.int32),
        pltpu.VMEM_SHARED((_NA, _RV), jnp.float32),
        pltpu.VMEM_SHARED((_NA, _AVP), jnp.float32),
        pltpu.VMEM((_NA, _RV), jnp.float32),
        pltpu.VMEM((_NA, _AVP), jnp.float32),
        pltpu.VMEM((_NA, _SPT), jnp.float32),
        pltpu.SemaphoreType.DMA,
        pltpu.SemaphoreType.DMA,
        pltpu.SemaphoreType.DMA,
        pltpu.SemaphoreType.DMA,
    ],
)
def _sc_kernel(p_hbm, rt_hbm, at_hbm, w_hbm, b_hbm, out_hbm,
               p_v, rt_v, at_v, w_v, b_v, chunk_v, idx_v,
               pr_sh, pa_sh, pr_v, pa_v, out_v,
               sem_p, sem_rt, sem_w, sem_b):
    _sc_body(p_hbm, rt_hbm, at_hbm, w_hbm, b_hbm, out_hbm,
             p_v, rt_v, at_v, w_v, b_v, chunk_v, idx_v,
             pr_sh, pa_sh, pr_v, pa_v, out_v,
             sem_p, sem_rt, sem_w, sem_b)


def kernel(agent_positions, goal_positions, radius_table, angle_table, W, b):
    pos = jnp.concatenate([agent_positions, goal_positions], axis=1).T
    out = _sc_kernel(pos, radius_table, angle_table, W, b)
    return out.T
